# Initial kernel scaffold; baseline (speedup 1.0000x reference)
import jax
import jax.numpy as jnp
from jax.experimental import pallas as pl


def kernel(mem_bank, embeddings, queries, mem_ids):
    raise NotImplementedError("write your pallas kernel here")


# speedup vs baseline: 1.0304x; 1.0304x over previous
import functools
import numpy as np
import jax
import jax.numpy as jnp
from jax import lax
from jax.experimental import pallas as pl
from jax.experimental.pallas import tpu as pltpu
from jax.experimental.pallas import tpu_sc as plsc

BANK = 8
BS = 32
NI = 300
D = 256
NPAD = 304
NV = NPAD // 16
CH = 16
NCHUNK = NPAD // CH
LANES = 16
DC = D // LANES


def _pe_table():
    inv_freq = (1.0 / (10000.0 ** (np.arange(0, D, 2, dtype=np.float32) / np.float32(D)))).astype(np.float32)
    pos = np.arange(BANK, dtype=np.float32)
    sin_inp = pos[:, None] * inv_freq[None, :]
    emb = np.stack((np.sin(sin_inp), np.cos(sin_inp)), axis=-1).reshape(BANK, D)
    return jnp.asarray(emb, dtype=jnp.float32)


@functools.partial(
    pl.kernel,
    mesh=plsc.VectorSubcoreMesh(core_axis_name="c", subcore_axis_name="s"),
    compiler_params=pltpu.CompilerParams(needs_layout_passes=False),
    out_type=jax.ShapeDtypeStruct((BS, NPAD, D), jnp.float32),
    scratch_types=[
        pltpu.VMEM((NPAD,), jnp.int32),
        pltpu.VMEM((NPAD,), jnp.int32),
    ] + [pltpu.VMEM((CH,), jnp.int32) for _ in range(BANK + 1)]
    + [
        pltpu.VMEM((BANK, CH, D), jnp.float32),
        pltpu.VMEM((CH, D), jnp.float32),
        pltpu.VMEM((CH, D), jnp.float32),
        pltpu.VMEM((BANK, D), jnp.float32),
        pltpu.SemaphoreType.DMA,
    ],
)
def _attn(mb_hbm, emb_hbm, q_hbm, ids_hbm, pe_hbm, out_hbm,
          ids_v, last_v, i0, i1, i2, i3, i4, i5, i6, i7, iq,
          g_v, q_v, o_v, pe_v, sem):
    idx_refs = [i0, i1, i2, i3, i4, i5, i6, i7]
    b = lax.axis_index("s") * 2 + lax.axis_index("c")

    pltpu.sync_copy(ids_hbm.at[b], ids_v)
    pltpu.sync_copy(pe_hbm, pe_v)

    iot = lax.iota(jnp.int32, LANES)
    zeros = jnp.zeros((LANES,), jnp.int32)

    for v in range(NV):
        last_v[pl.ds(v * LANES, LANES)] = zeros

    for v in range(NV):
        idvec = ids_v[pl.ds(v * LANES, LANES)]
        ivec = iot + (v * LANES)
        plsc.store_scatter(last_v, [idvec], ivec, mask=ivec < NI)

    def _chunk(t, carry):
        n0 = t * CH
        idvec = ids_v[pl.ds(n0, CH)]
        for k in range(BANK - 1):
            idx_refs[k][...] = idvec + (k * BS + b) * NI
        idx_refs[BANK - 1][...] = plsc.load_gather(last_v, [idvec]) + b * NI
        iq[...] = jnp.minimum(n0 + iot + b * NI, BS * NI - 1)

        copies = []
        for k in range(BANK - 1):
            copies.append(pltpu.async_copy(
                mb_hbm.at[idx_refs[k]], g_v.at[k], sem))
        copies.append(pltpu.async_copy(
            emb_hbm.at[idx_refs[BANK - 1]], g_v.at[BANK - 1], sem))
        copies.append(pltpu.async_copy(q_hbm.at[iq], q_v, sem))
        for cpy in copies:
            cpy.wait()

        def _row(n, c2):
            qr = [q_v[n, pl.ds(c * LANES, LANES)] for c in range(DC)]
            s = jnp.full((LANES,), -1e30, jnp.float32)
            for k in range(BANK):
                sl = pl.ds(0, LANES)
                acc = (g_v[k, n, sl] + pe_v[k, sl]) * qr[0]
                for c in range(1, DC):
                    sl = pl.ds(c * LANES, LANES)
                    acc = acc + (g_v[k, n, sl] + pe_v[k, sl]) * qr[c]
                s = jnp.where(iot == k, jnp.sum(acc), s)
            s = s * jnp.float32(1.0 / 16.0)
            e = jnp.exp(s - jnp.max(s))
            w = e / jnp.sum(e)
            wk = [w[k] for k in range(BANK)]
            for c in range(DC):
                sl = pl.ds(c * LANES, LANES)
                o = wk[0] * (g_v[0, n, sl] + pe_v[0, sl])
                for k in range(1, BANK):
                    o = o + wk[k] * (g_v[k, n, sl] + pe_v[k, sl])
                o_v[n, sl] = o
            return c2
        lax.fori_loop(0, CH, _row, 0)

        pltpu.sync_copy(o_v, out_hbm.at[b, pl.ds(n0, CH)])
        return carry

    lax.fori_loop(0, NCHUNK, _chunk, 0)


def kernel(mem_bank, embeddings, queries, mem_ids):
    mb_flat = mem_bank.reshape(BANK * BS * NI, D)
    emb_flat = embeddings.reshape(BS * NI, D)
    q_flat = queries.reshape(BS * NI, D)
    ids = mem_ids.astype(jnp.int32)
    ids_pad = jnp.concatenate(
        [ids, jnp.zeros((BS, NPAD - NI), jnp.int32)], axis=1)
    out = _attn(mb_flat, emb_flat, q_flat, ids_pad, _pe_table())
    return out[:, :NI, :]


# speedup vs baseline: 1.2335x; 1.1971x over previous
import functools
import numpy as np
import jax
import jax.numpy as jnp
from jax import lax
from jax.experimental import pallas as pl
from jax.experimental.pallas import tpu as pltpu
from jax.experimental.pallas import tpu_sc as plsc

BANK = 8
BS = 32
NI = 300
D = 256
NPAD = 304
NV = NPAD // 16
CH = 16
NCHUNK = NPAD // CH
LANES = 16
DC = D // LANES


def _pe_table():
    inv_freq = (1.0 / (10000.0 ** (np.arange(0, D, 2, dtype=np.float32) / np.float32(D)))).astype(np.float32)
    pos = np.arange(BANK, dtype=np.float32)
    sin_inp = pos[:, None] * inv_freq[None, :]
    emb = np.stack((np.sin(sin_inp), np.cos(sin_inp)), axis=-1).reshape(BANK, D)
    return jnp.asarray(emb, dtype=jnp.float32)


@functools.partial(
    pl.kernel,
    mesh=plsc.VectorSubcoreMesh(core_axis_name="c", subcore_axis_name="s"),
    compiler_params=pltpu.CompilerParams(needs_layout_passes=False),
    out_type=jax.ShapeDtypeStruct((BS, NPAD, D), jnp.float32),
    scratch_types=[
        pltpu.VMEM((NPAD,), jnp.int32),
        pltpu.VMEM((NPAD,), jnp.int32),
        pltpu.VMEM((CH,), jnp.int32),
        pltpu.VMEM((CH,), jnp.int32),
        pltpu.VMEM((CH,), jnp.int32),
        pltpu.VMEM((BANK, CH, D), jnp.float32),
        pltpu.VMEM((CH, D), jnp.float32),
        pltpu.VMEM((CH, D), jnp.float32),
        pltpu.VMEM((BANK, D), jnp.float32),
        pltpu.SemaphoreType.DMA,
    ],
)
def _attn(mb_hbm, emb_hbm, q_hbm, ids_hbm, pe_hbm, out_hbm,
          ids_v, last_v, ii, il, iq, g_v, q_v, o_v, pe_v, sem):
    b = lax.axis_index("s") * 2 + lax.axis_index("c")

    pltpu.sync_copy(ids_hbm.at[b], ids_v)
    pltpu.sync_copy(pe_hbm, pe_v)

    iot = lax.iota(jnp.int32, LANES)
    zeros = jnp.zeros((LANES,), jnp.int32)

    for v in range(NV):
        last_v[pl.ds(v * LANES, LANES)] = zeros

    for v in range(NV):
        idvec = ids_v[pl.ds(v * LANES, LANES)]
        ivec = iot + (v * LANES)
        plsc.store_scatter(last_v, [idvec], ivec, mask=ivec < NI)

    def _chunk(t, carry):
        n0 = t * CH
        idvec = ids_v[pl.ds(n0, CH)]
        ii[...] = idvec
        il[...] = plsc.load_gather(last_v, [idvec])
        iq[...] = jnp.minimum(n0 + iot, NI - 1)

        copies = []
        for k in range(BANK - 1):
            copies.append(pltpu.async_copy(
                mb_hbm.at[k, b].at[ii], g_v.at[k], sem))
        copies.append(pltpu.async_copy(
            emb_hbm.at[b].at[il], g_v.at[BANK - 1], sem))
        copies.append(pltpu.async_copy(q_hbm.at[b].at[iq], q_v, sem))
        for cpy in copies:
            cpy.wait()

        def _row(n, c2):
            qr = [q_v[n, pl.ds(c * LANES, LANES)] for c in range(DC)]
            s = jnp.full((LANES,), -1e30, jnp.float32)
            for k in range(BANK):
                sl = pl.ds(0, LANES)
                acc = (g_v[k, n, sl] + pe_v[k, sl]) * qr[0]
                for c in range(1, DC):
                    sl = pl.ds(c * LANES, LANES)
                    acc = acc + (g_v[k, n, sl] + pe_v[k, sl]) * qr[c]
                s = jnp.where(iot == k, jnp.sum(acc), s)
            s = s * jnp.float32(1.0 / 16.0)
            e = jnp.exp(s - jnp.max(s))
            w = e / jnp.sum(e)
            wk = [w[k] for k in range(BANK)]
            for c in range(DC):
                sl = pl.ds(c * LANES, LANES)
                o = wk[0] * (g_v[0, n, sl] + pe_v[0, sl])
                for k in range(1, BANK):
                    o = o + wk[k] * (g_v[k, n, sl] + pe_v[k, sl])
                o_v[n, sl] = o
            return c2
        lax.fori_loop(0, CH, _row, 0)

        pltpu.sync_copy(o_v, out_hbm.at[b, pl.ds(n0, CH)])
        return carry

    lax.fori_loop(0, NCHUNK, _chunk, 0)


def kernel(mem_bank, embeddings, queries, mem_ids):
    ids = mem_ids.astype(jnp.int32)
    ids_pad = jnp.concatenate(
        [ids, jnp.zeros((BS, NPAD - NI), jnp.int32)], axis=1)
    out = _attn(mem_bank, embeddings, queries, ids_pad, _pe_table())
    return out[:, :NI, :]


# speedup vs baseline: 1.2348x; 1.0011x over previous
import functools
import numpy as np
import jax
import jax.numpy as jnp
from jax import lax
from jax.experimental import pallas as pl
from jax.experimental.pallas import tpu as pltpu
from jax.experimental.pallas import tpu_sc as plsc

BANK = 8
BS = 32
NI = 300
D = 256
NPAD = 304
NV = NPAD // 16
CH = 16
NCHUNK = NPAD // CH
LANES = 16
DC = D // LANES


def _pe_table():
    inv_freq = (1.0 / (10000.0 ** (np.arange(0, D, 2, dtype=np.float32) / np.float32(D)))).astype(np.float32)
    pos = np.arange(BANK, dtype=np.float32)
    sin_inp = pos[:, None] * inv_freq[None, :]
    emb = np.stack((np.sin(sin_inp), np.cos(sin_inp)), axis=-1).reshape(BANK, D)
    return jnp.asarray(emb, dtype=jnp.float32)


@functools.partial(
    pl.kernel,
    mesh=plsc.VectorSubcoreMesh(core_axis_name="c", subcore_axis_name="s"),
    compiler_params=pltpu.CompilerParams(
        needs_layout_passes=False, use_tc_tiling_on_sc=True),
    out_type=jax.ShapeDtypeStruct((BS, NPAD, D), jnp.float32),
    scratch_types=[
        pltpu.VMEM((NPAD,), jnp.int32),
        pltpu.VMEM((NPAD,), jnp.int32),
        pltpu.VMEM((CH,), jnp.int32),
        pltpu.VMEM((CH,), jnp.int32),
        pltpu.VMEM((CH,), jnp.int32),
        pltpu.VMEM((BANK, CH, D), jnp.float32),
        pltpu.VMEM((CH, D), jnp.float32),
        pltpu.VMEM((CH, D), jnp.float32),
        pltpu.VMEM((BANK, D), jnp.float32),
        pltpu.SemaphoreType.DMA,
    ],
)
def _attn(mb_hbm, emb_hbm, q_hbm, ids_hbm, pe_hbm, out_hbm,
          ids_v, last_v, ii, il, iq, g_v, q_v, o_v, pe_v, sem):
    b = lax.axis_index("s") * 2 + lax.axis_index("c")

    pltpu.sync_copy(ids_hbm.at[b], ids_v)
    pltpu.sync_copy(pe_hbm, pe_v)

    iot = lax.iota(jnp.int32, LANES)
    zeros = jnp.zeros((LANES,), jnp.int32)

    for v in range(NV):
        last_v[pl.ds(v * LANES, LANES)] = zeros

    for v in range(NV):
        idvec = ids_v[pl.ds(v * LANES, LANES)]
        ivec = iot + (v * LANES)
        plsc.store_scatter(last_v, [idvec], ivec, mask=ivec < NI)

    def _chunk(t, carry):
        n0 = t * CH
        idvec = ids_v[pl.ds(n0, CH)]
        ii[...] = idvec
        il[...] = plsc.load_gather(last_v, [idvec])
        iq[...] = jnp.minimum(n0 + iot, NI - 1)

        copies = []
        for k in range(BANK - 1):
            copies.append(pltpu.async_copy(
                mb_hbm.at[k, b].at[ii], g_v.at[k], sem))
        copies.append(pltpu.async_copy(
            emb_hbm.at[b].at[il], g_v.at[BANK - 1], sem))
        copies.append(pltpu.async_copy(q_hbm.at[b].at[iq], q_v, sem))
        for cpy in copies:
            cpy.wait()

        def _row(n, c2):
            qr = [q_v[n, pl.ds(c * LANES, LANES)] for c in range(DC)]
            s = jnp.full((LANES,), -1e30, jnp.float32)
            for k in range(BANK):
                sl = pl.ds(0, LANES)
                acc = (g_v[k, n, sl] + pe_v[k, sl]) * qr[0]
                for c in range(1, DC):
                    sl = pl.ds(c * LANES, LANES)
                    acc = acc + (g_v[k, n, sl] + pe_v[k, sl]) * qr[c]
                s = jnp.where(iot == k, jnp.sum(acc), s)
            s = s * jnp.float32(1.0 / 16.0)
            e = jnp.exp(s - jnp.max(s))
            w = e / jnp.sum(e)
            wk = [w[k] for k in range(BANK)]
            for c in range(DC):
                sl = pl.ds(c * LANES, LANES)
                o = wk[0] * (g_v[0, n, sl] + pe_v[0, sl])
                for k in range(1, BANK):
                    o = o + wk[k] * (g_v[k, n, sl] + pe_v[k, sl])
                o_v[n, sl] = o
            return c2
        lax.fori_loop(0, CH, _row, 0)

        pltpu.sync_copy(o_v, out_hbm.at[b, pl.ds(n0, CH)])
        return carry

    lax.fori_loop(0, NCHUNK, _chunk, 0)


def kernel(mem_bank, embeddings, queries, mem_ids):
    ids = mem_ids.astype(jnp.int32)
    ids_pad = jnp.concatenate(
        [ids, jnp.zeros((BS, NPAD - NI), jnp.int32)], axis=1)
    out = _attn(mem_bank, embeddings, queries, ids_pad, _pe_table())
    return out[:, :NI, :]
